# Initial kernel scaffold; baseline (speedup 1.0000x reference)
import functools

import jax
import jax.numpy as jnp
from jax import lax
from jax.experimental import pallas as pl
from jax.experimental.pallas import tpu as pltpu
from jax.experimental.pallas import tpu_sc as plsc

N = 10000
F = 128
C = 40
NP = 10240
NC = 2
NS = 16
NW = NC * NS
K = 128
CH = 79
EPT = CH * K
E_PAD = NW * EPT
RPT = NP // NS


def _sc_degrees_body(src_hbm, dst_hbm, degout_hbm, degin_hbm,
                     sidx, didx, valbuf, degout_sh, degin_sh):
    cid = lax.axis_index("c")
    sid = lax.axis_index("s")
    wid = sid * NC + cid
    base = sid * RPT

    pltpu.sync_copy(src_hbm.at[pl.ds(wid * CH, CH)], sidx)
    pltpu.sync_copy(dst_hbm.at[pl.ds(wid * CH, CH)], didx)

    for i in range(K // 16):
        valbuf[pl.ds(i * 16, 16)] = jnp.zeros((16,), jnp.float32)
    for i in range(RPT // K):
        pltpu.sync_copy(valbuf, degout_sh.at[pl.ds(base + i * K, K)])
        pltpu.sync_copy(valbuf, degin_sh.at[pl.ds(base + i * K, K)])
    plsc.subcore_barrier()

    for i in range(K // 16):
        valbuf[pl.ds(i * 16, 16)] = jnp.ones((16,), jnp.float32)

    def body(j, carry):
        pltpu.sync_copy(valbuf, degout_sh.at[sidx.at[j]], add=True)
        pltpu.sync_copy(valbuf, degin_sh.at[didx.at[j]], add=True)
        return carry

    lax.fori_loop(0, CH, body, 0)
    plsc.subcore_barrier()

    pltpu.sync_copy(degout_sh.at[pl.ds(base, RPT)],
                    degout_hbm.at[cid, pl.ds(base, RPT)])
    pltpu.sync_copy(degin_sh.at[pl.ds(base, RPT)],
                    degin_hbm.at[cid, pl.ds(base, RPT)])


_sc_degrees = functools.partial(
    pl.kernel,
    _sc_degrees_body,
    out_type=(jax.ShapeDtypeStruct((NC, NP), jnp.float32),
              jax.ShapeDtypeStruct((NC, NP), jnp.float32)),
    mesh=plsc.VectorSubcoreMesh(core_axis_name="c", subcore_axis_name="s"),
    scratch_types=[
        pltpu.VMEM((CH, K), jnp.int32),
        pltpu.VMEM((CH, K), jnp.int32),
        pltpu.VMEM((K,), jnp.float32),
        pltpu.VMEM_SHARED((NP,), jnp.float32),
        pltpu.VMEM_SHARED((NP,), jnp.float32),
    ],
)()


def _sc_main_body(y_hbm, src_hbm, dst_hbm, nd_hbm, agg_hbm, w_hbm,
                  sidx, didx, rows, vals, gsem, agg_sh, w_sh):
    cid = lax.axis_index("c")
    sid = lax.axis_index("s")
    wid = sid * NC + cid
    base = sid * RPT

    pltpu.sync_copy(src_hbm.at[pl.ds(wid * CH, CH)], sidx)
    pltpu.sync_copy(dst_hbm.at[pl.ds(wid * CH, CH)], didx)

    def zrow(t, carry):
        rows[t // 8, pl.ds((t % 8) * 16, 16)] = jnp.zeros((16,), jnp.float32)
        return carry

    lax.fori_loop(0, K * 8, zrow, 0)
    for i in range(K // 16):
        vals[pl.ds(i * 16, 16)] = jnp.zeros((16,), jnp.float32)
    for i in range(RPT // K):
        pltpu.sync_copy(rows, agg_sh.at[pl.ds(base + i * K, K)])
        pltpu.sync_copy(vals, w_sh.at[pl.ds(base + i * K, K)])
    plsc.subcore_barrier()

    def body(j, carry):
        pltpu.async_copy(y_hbm.at[sidx.at[j]], rows, gsem).wait()
        pltpu.sync_copy(rows, agg_sh.at[didx.at[j]], add=True)
        pltpu.async_copy(nd_hbm.at[didx.at[j]], vals, gsem).wait()
        pltpu.sync_copy(vals, w_sh.at[sidx.at[j]], add=True)
        return carry

    lax.fori_loop(0, CH, body, 0)
    plsc.subcore_barrier()

    for i in range(RPT // K):
        pltpu.sync_copy(agg_sh.at[pl.ds(base + i * K, K)],
                        agg_hbm.at[cid, pl.ds(base + i * K, K)])
        pltpu.sync_copy(w_sh.at[pl.ds(base + i * K, K)],
                        w_hbm.at[cid, pl.ds(base + i * K, K)])


_sc_main = functools.partial(
    pl.kernel,
    _sc_main_body,
    out_type=(jax.ShapeDtypeStruct((NC, NP, F), jnp.float32),
              jax.ShapeDtypeStruct((NC, NP), jnp.float32)),
    mesh=plsc.VectorSubcoreMesh(core_axis_name="c", subcore_axis_name="s"),
    scratch_types=[
        pltpu.VMEM((CH, K), jnp.int32),
        pltpu.VMEM((CH, K), jnp.int32),
        pltpu.VMEM((K, F), jnp.float32),
        pltpu.VMEM((K,), jnp.float32),
        pltpu.SemaphoreType.DMA,
        pltpu.VMEM_SHARED((NP, F), jnp.float32),
        pltpu.VMEM_SHARED((NP,), jnp.float32),
    ],
)()


def _tc_prep_body(x_ref, do0, do1, di0, di1, w1_ref, y_ref, ns_ref, nd_ref):
    ns = lax.rsqrt(jnp.maximum(do0[...] + do1[...], 1.0))
    nd = lax.rsqrt(jnp.maximum(di0[...] + di1[...], 1.0))
    xs = x_ref[...] * ns
    y_ref[...] = jnp.dot(xs, w1_ref[...], preferred_element_type=jnp.float32)
    ns_ref[...] = ns
    nd_ref[...] = nd


_tc_prep = pl.pallas_call(
    _tc_prep_body,
    out_shape=(jax.ShapeDtypeStruct((NP, F), jnp.float32),
               jax.ShapeDtypeStruct((NP, 1), jnp.float32),
               jax.ShapeDtypeStruct((NP, 1), jnp.float32)),
)


def _tc_final_body(a0, a1, ns, nd, w0, w1, b1r, w2p, b2p, out_ref):
    agg = a0[...] + a1[...]
    h1 = jnp.maximum(nd[...] * agg + b1r[...], 0.0)
    h2s = h1 * ns[...]
    w = w0[...] + w1[...]
    rid = lax.broadcasted_iota(jnp.int32, (NP, 1), 0)
    wm = jnp.where(rid < N, w, 0.0)
    z = jnp.sum(h2s * wm, axis=0, keepdims=True)
    out_ref[...] = (jnp.dot(z * (1.0 / N), w2p[...],
                            preferred_element_type=jnp.float32) + b2p[...])


_tc_final = pl.pallas_call(
    _tc_final_body,
    out_shape=jax.ShapeDtypeStruct((1, F), jnp.float32),
)


def kernel(in_feat, edge_index, W1, b1, W2, b2):
    e = edge_index.shape[1]
    src = jnp.concatenate(
        [edge_index[0], jnp.full((E_PAD - e,), N, jnp.int32)]).reshape(NW * CH, K)
    dst = jnp.concatenate(
        [edge_index[1], jnp.full((E_PAD - e,), N, jnp.int32)]).reshape(NW * CH, K)
    x_p = jnp.pad(in_feat, ((0, NP - N), (0, 0)))

    degout_p, degin_p = _sc_degrees(src, dst)

    y, ns, nd = _tc_prep(x_p,
                         degout_p[0].reshape(NP, 1), degout_p[1].reshape(NP, 1),
                         degin_p[0].reshape(NP, 1), degin_p[1].reshape(NP, 1),
                         W1)

    agg_p, w_p = _sc_main(y, src, dst, nd.reshape(NP))

    w2p = jnp.pad(W2, ((0, 0), (0, F - C)))
    b2p = jnp.pad(b2, (0, F - C)).reshape(1, F)
    out = _tc_final(agg_p[0], agg_p[1], ns, nd,
                    w_p[0].reshape(NP, 1), w_p[1].reshape(NP, 1),
                    b1.reshape(1, F), w2p, b2p)
    return out[0, :C]


# speedup vs baseline: 5.5395x; 5.5395x over previous
import functools

import jax
import jax.numpy as jnp
from jax import lax
from jax.experimental import pallas as pl
from jax.experimental.pallas import tpu as pltpu
from jax.experimental.pallas import tpu_sc as plsc

N = 10000
F = 128
C = 40
NP = 10240
NC = 2
NS = 16
NW = NC * NS
K = 128
CH = 80
EPT = CH * K
E_PAD = NW * EPT
RPT = NP // NS


def _sc_degrees_body(src_hbm, dst_hbm, degout_hbm, degin_hbm,
                     sidx, didx, valbuf, degout_sh, degin_sh):
    cid = lax.axis_index("c")
    sid = lax.axis_index("s")
    wid = sid * NC + cid
    base = sid * RPT

    pltpu.sync_copy(src_hbm.at[pl.ds(wid * CH, CH)], sidx)
    pltpu.sync_copy(dst_hbm.at[pl.ds(wid * CH, CH)], didx)

    for i in range(K // 16):
        valbuf[pl.ds(i * 16, 16)] = jnp.zeros((16,), jnp.float32)
    for i in range(RPT // K):
        pltpu.sync_copy(valbuf, degout_sh.at[pl.ds(base + i * K, K)])
        pltpu.sync_copy(valbuf, degin_sh.at[pl.ds(base + i * K, K)])
    plsc.subcore_barrier()

    for i in range(K // 16):
        valbuf[pl.ds(i * 16, 16)] = jnp.ones((16,), jnp.float32)

    def body(j, carry):
        pltpu.sync_copy(valbuf, degout_sh.at[sidx.at[j]], add=True)
        pltpu.sync_copy(valbuf, degin_sh.at[didx.at[j]], add=True)
        return carry

    lax.fori_loop(0, CH, body, 0)
    plsc.subcore_barrier()

    pltpu.sync_copy(degout_sh.at[pl.ds(base, RPT)],
                    degout_hbm.at[pl.ds(cid * NP + base, RPT)])
    pltpu.sync_copy(degin_sh.at[pl.ds(base, RPT)],
                    degin_hbm.at[pl.ds(cid * NP + base, RPT)])


_sc_degrees = functools.partial(
    pl.kernel,
    _sc_degrees_body,
    out_type=(jax.ShapeDtypeStruct((NC * NP,), jnp.float32),
              jax.ShapeDtypeStruct((NC * NP,), jnp.float32)),
    mesh=plsc.VectorSubcoreMesh(core_axis_name="c", subcore_axis_name="s"),
    scratch_types=[
        pltpu.VMEM((CH, K), jnp.int32),
        pltpu.VMEM((CH, K), jnp.int32),
        pltpu.VMEM((K,), jnp.float32),
        pltpu.VMEM_SHARED((NP,), jnp.float32),
        pltpu.VMEM_SHARED((NP,), jnp.float32),
    ],
)()


def _sc_main_body(y_hbm, src_hbm, dst_hbm, nd_hbm, agg_hbm, w_hbm,
                  sidx, didx, rows, vals, gsem, agg_sh, w_sh):
    cid = lax.axis_index("c")
    sid = lax.axis_index("s")
    wid = sid * NC + cid
    base = sid * RPT

    pltpu.sync_copy(src_hbm.at[pl.ds(wid * CH, CH)], sidx)
    pltpu.sync_copy(dst_hbm.at[pl.ds(wid * CH, CH)], didx)

    def zrow(t, carry):
        rows[t // 8, pl.ds((t % 8) * 16, 16)] = jnp.zeros((16,), jnp.float32)
        return carry

    lax.fori_loop(0, K * 8, zrow, 0)
    for i in range(K // 16):
        vals[pl.ds(i * 16, 16)] = jnp.zeros((16,), jnp.float32)
    for i in range(RPT // K):
        pltpu.sync_copy(rows, agg_sh.at[pl.ds(base + i * K, K)])
        pltpu.sync_copy(vals, w_sh.at[pl.ds(base + i * K, K)])
    plsc.subcore_barrier()

    def body(j, carry):
        pltpu.async_copy(y_hbm.at[sidx.at[j]], rows, gsem).wait()
        pltpu.sync_copy(rows, agg_sh.at[didx.at[j]], add=True)
        pltpu.async_copy(nd_hbm.at[didx.at[j]], vals, gsem).wait()
        pltpu.sync_copy(vals, w_sh.at[sidx.at[j]], add=True)
        return carry

    lax.fori_loop(0, CH, body, 0)
    plsc.subcore_barrier()

    for i in range(RPT // K):
        pltpu.sync_copy(agg_sh.at[pl.ds(base + i * K, K)],
                        agg_hbm.at[cid, pl.ds(base + i * K, K)])
        pltpu.sync_copy(w_sh.at[pl.ds(base + i * K, K)],
                        w_hbm.at[pl.ds(cid * NP + base + i * K, K)])


_sc_main = functools.partial(
    pl.kernel,
    _sc_main_body,
    out_type=(jax.ShapeDtypeStruct((NC, NP, F), jnp.float32),
              jax.ShapeDtypeStruct((NC * NP,), jnp.float32)),
    mesh=plsc.VectorSubcoreMesh(core_axis_name="c", subcore_axis_name="s"),
    scratch_types=[
        pltpu.VMEM((CH, K), jnp.int32),
        pltpu.VMEM((CH, K), jnp.int32),
        pltpu.VMEM((K, F), jnp.float32),
        pltpu.VMEM((K,), jnp.float32),
        pltpu.SemaphoreType.DMA,
        pltpu.VMEM_SHARED((NP, F), jnp.float32),
        pltpu.VMEM_SHARED((NP,), jnp.float32),
    ],
)()


def _tc_prep_body(x_ref, do0, do1, di0, di1, w1_ref, y_ref, ns_ref, nd_ref):
    ns = lax.rsqrt(jnp.maximum(do0[...] + do1[...], 1.0))
    nd = lax.rsqrt(jnp.maximum(di0[...] + di1[...], 1.0))
    xs = x_ref[...] * ns
    y_ref[...] = jnp.dot(xs, w1_ref[...], preferred_element_type=jnp.float32)
    ns_ref[...] = ns
    nd_ref[...] = nd


_tc_prep = pl.pallas_call(
    _tc_prep_body,
    out_shape=(jax.ShapeDtypeStruct((NP, F), jnp.float32),
               jax.ShapeDtypeStruct((NP, 1), jnp.float32),
               jax.ShapeDtypeStruct((NP, 1), jnp.float32)),
)


def _tc_final_body(a0, a1, ns, nd, w0, w1, b1r, w2p, b2p, out_ref):
    agg = a0[...] + a1[...]
    h1 = jnp.maximum(nd[...] * agg + b1r[...], 0.0)
    h2s = h1 * ns[...]
    w = w0[...] + w1[...]
    rid = lax.broadcasted_iota(jnp.int32, (NP, 1), 0)
    wm = jnp.where(rid < N, w, 0.0)
    z = jnp.sum(h2s * wm, axis=0, keepdims=True)
    out_ref[...] = (jnp.dot(z * (1.0 / N), w2p[...],
                            preferred_element_type=jnp.float32) + b2p[...])


_tc_final = pl.pallas_call(
    _tc_final_body,
    out_shape=jax.ShapeDtypeStruct((1, F), jnp.float32),
)


def kernel(in_feat, edge_index, W1, b1, W2, b2):
    e = edge_index.shape[1]
    src = jnp.concatenate(
        [edge_index[0], jnp.full((E_PAD - e,), N, jnp.int32)]).reshape(NW * CH, K)
    dst = jnp.concatenate(
        [edge_index[1], jnp.full((E_PAD - e,), N, jnp.int32)]).reshape(NW * CH, K)
    x_p = jnp.pad(in_feat, ((0, NP - N), (0, 0)))

    degout_p, degin_p = _sc_degrees(src, dst)
    degout_p = degout_p.reshape(NC, NP)
    degin_p = degin_p.reshape(NC, NP)

    y, ns, nd = _tc_prep(x_p,
                         degout_p[0].reshape(NP, 1), degout_p[1].reshape(NP, 1),
                         degin_p[0].reshape(NP, 1), degin_p[1].reshape(NP, 1),
                         W1)

    agg_p, w_p = _sc_main(y, src, dst, nd.reshape(NP))
    w_p = w_p.reshape(NC, NP)

    w2p = jnp.pad(W2, ((0, 0), (0, F - C)))
    b2p = jnp.pad(b2, (0, F - C)).reshape(1, F)
    out = _tc_final(agg_p[0], agg_p[1], ns, nd,
                    w_p[0].reshape(NP, 1), w_p[1].reshape(NP, 1),
                    b1.reshape(1, F), w2p, b2p)
    return out[0, :C]


# speedup vs baseline: 6.7523x; 1.2189x over previous
import functools

import jax
import jax.numpy as jnp
from jax import lax
from jax.experimental import pallas as pl
from jax.experimental.pallas import tpu as pltpu
from jax.experimental.pallas import tpu_sc as plsc

N = 10000
F = 128
C = 40
NP = 10240
NC = 2
NS = 16
NW = NC * NS
K = 128
CH = 80
EPT = CH * K
E_PAD = NW * EPT
RPT = NP // NS


def _sc_degrees_body(src_hbm, dst_hbm, degout_hbm, degin_hbm,
                     sidx, didx, valbuf, degout_sh, degin_sh):
    cid = lax.axis_index("c")
    sid = lax.axis_index("s")
    wid = sid * NC + cid
    base = sid * RPT

    pltpu.sync_copy(src_hbm.at[pl.ds(wid * CH, CH)], sidx)
    pltpu.sync_copy(dst_hbm.at[pl.ds(wid * CH, CH)], didx)

    for i in range(K // 16):
        valbuf[pl.ds(i * 16, 16)] = jnp.zeros((16,), jnp.float32)
    for i in range(RPT // K):
        pltpu.sync_copy(valbuf, degout_sh.at[pl.ds(base + i * K, K)])
        pltpu.sync_copy(valbuf, degin_sh.at[pl.ds(base + i * K, K)])
    plsc.subcore_barrier()

    for i in range(K // 16):
        valbuf[pl.ds(i * 16, 16)] = jnp.ones((16,), jnp.float32)

    def body(j, carry):
        pltpu.sync_copy(valbuf, degout_sh.at[sidx.at[j]], add=True)
        pltpu.sync_copy(valbuf, degin_sh.at[didx.at[j]], add=True)
        return carry

    lax.fori_loop(0, CH, body, 0)
    plsc.subcore_barrier()

    pltpu.sync_copy(degout_sh.at[pl.ds(base, RPT)],
                    degout_hbm.at[pl.ds(cid * NP + base, RPT)])
    pltpu.sync_copy(degin_sh.at[pl.ds(base, RPT)],
                    degin_hbm.at[pl.ds(cid * NP + base, RPT)])


_sc_degrees = functools.partial(
    pl.kernel,
    _sc_degrees_body,
    out_type=(jax.ShapeDtypeStruct((NC * NP,), jnp.float32),
              jax.ShapeDtypeStruct((NC * NP,), jnp.float32)),
    mesh=plsc.VectorSubcoreMesh(core_axis_name="c", subcore_axis_name="s"),
    scratch_types=[
        pltpu.VMEM((CH, K), jnp.int32),
        pltpu.VMEM((CH, K), jnp.int32),
        pltpu.VMEM((K,), jnp.float32),
        pltpu.VMEM_SHARED((NP,), jnp.float32),
        pltpu.VMEM_SHARED((NP,), jnp.float32),
    ],
)()


def _sc_main_body(y_hbm, src_hbm, dst_hbm, nd_hbm, agg_hbm, w_hbm,
                  sidx, didx, rows, vals, isem0, isem1, gsem0, gsem1, vsem,
                  agg_sh, w_sh):
    cid = lax.axis_index("c")
    sid = lax.axis_index("s")
    wid = sid * NC + cid
    base = sid * RPT
    isems = (isem0, isem1)
    gsems = (gsem0, gsem1)

    def fire_idx(j, b, sem):
        pltpu.async_copy(src_hbm.at[wid * CH + j], sidx.at[b], sem)
        pltpu.async_copy(dst_hbm.at[wid * CH + j], didx.at[b], sem)

    def wait_idx(j, b, sem):
        pltpu.make_async_copy(src_hbm.at[wid * CH + j], sidx.at[b], sem).wait()
        pltpu.make_async_copy(dst_hbm.at[wid * CH + j], didx.at[b], sem).wait()

    def zrow(t, carry):
        rows[0, t // 8, pl.ds((t % 8) * 16, 16)] = jnp.zeros((16,), jnp.float32)
        return carry

    lax.fori_loop(0, K * 8, zrow, 0)
    for i in range(RPT // K):
        pltpu.sync_copy(rows.at[0], agg_sh.at[pl.ds(base + i * K, K)])
    for i in range(RPT // F):
        pltpu.sync_copy(rows.at[0, 0], w_sh.at[pl.ds(base + i * F, F)])
    plsc.subcore_barrier()

    fire_idx(0, 0, isem0)
    fire_idx(1, 1, isem1)
    wait_idx(0, 0, isem0)
    pltpu.async_copy(y_hbm.at[sidx.at[0]], rows.at[0], gsem0)

    def body(p, carry):
        for b in range(2):
            j = p * 2 + b
            cond_next = (j + 1 < CH)
            cond_pref = (j + 2 < CH)

            @pl.when(cond_next)
            def _():
                wait_idx(j + 1, 1 - b, isems[1 - b])
                pltpu.async_copy(y_hbm.at[sidx.at[1 - b]], rows.at[1 - b],
                                 gsems[1 - b])

            pltpu.async_copy(nd_hbm.at[didx.at[b]], vals, vsem)
            pltpu.make_async_copy(y_hbm.at[sidx.at[b]], rows.at[b],
                                  gsems[b]).wait()
            pltpu.sync_copy(rows.at[b], agg_sh.at[didx.at[b]], add=True)
            pltpu.make_async_copy(nd_hbm.at[didx.at[b]], vals, vsem).wait()
            pltpu.sync_copy(vals, w_sh.at[sidx.at[b]], add=True)

            @pl.when(cond_pref)
            def _():
                fire_idx(j + 2, b, isems[b])
        return carry

    lax.fori_loop(0, CH // 2, body, 0)
    plsc.subcore_barrier()

    for i in range(RPT // K):
        pltpu.sync_copy(agg_sh.at[pl.ds(base + i * K, K)],
                        agg_hbm.at[cid, pl.ds(base + i * K, K)])
    for i in range(RPT // F):
        pltpu.sync_copy(w_sh.at[pl.ds(base + i * F, F)],
                        w_hbm.at[pl.ds(cid * NP + base + i * F, F)])


_sc_main = functools.partial(
    pl.kernel,
    _sc_main_body,
    out_type=(jax.ShapeDtypeStruct((NC, NP, F), jnp.float32),
              jax.ShapeDtypeStruct((NC * NP,), jnp.float32)),
    mesh=plsc.VectorSubcoreMesh(core_axis_name="c", subcore_axis_name="s"),
    scratch_types=[
        pltpu.VMEM((2, K), jnp.int32),
        pltpu.VMEM((2, K), jnp.int32),
        pltpu.VMEM((2, K, F), jnp.float32),
        pltpu.VMEM((K,), jnp.float32),
        pltpu.SemaphoreType.DMA,
        pltpu.SemaphoreType.DMA,
        pltpu.SemaphoreType.DMA,
        pltpu.SemaphoreType.DMA,
        pltpu.SemaphoreType.DMA,
        pltpu.VMEM_SHARED((NP, F), jnp.float32),
        pltpu.VMEM_SHARED((NP,), jnp.float32),
    ],
)()


def _tc_prep_body(x_ref, do0, do1, di0, di1, w1_ref, y_ref, ns_ref, nd_ref):
    ns = lax.rsqrt(jnp.maximum(do0[...] + do1[...], 1.0))
    nd = lax.rsqrt(jnp.maximum(di0[...] + di1[...], 1.0))
    xs = x_ref[...] * ns
    y_ref[...] = jnp.dot(xs, w1_ref[...], preferred_element_type=jnp.float32)
    ns_ref[...] = ns
    nd_ref[...] = nd


_tc_prep = pl.pallas_call(
    _tc_prep_body,
    out_shape=(jax.ShapeDtypeStruct((NP, F), jnp.float32),
               jax.ShapeDtypeStruct((NP, 1), jnp.float32),
               jax.ShapeDtypeStruct((NP, 1), jnp.float32)),
)


def _tc_final_body(a0, a1, ns, nd, w0, w1, b1r, w2p, b2p, out_ref):
    agg = a0[...] + a1[...]
    h1 = jnp.maximum(nd[...] * agg + b1r[...], 0.0)
    h2s = h1 * ns[...]
    w = w0[...] + w1[...]
    rid = lax.broadcasted_iota(jnp.int32, (NP, 1), 0)
    wm = jnp.where(rid < N, w, 0.0)
    z = jnp.sum(h2s * wm, axis=0, keepdims=True)
    out_ref[...] = (jnp.dot(z * (1.0 / N), w2p[...],
                            preferred_element_type=jnp.float32) + b2p[...])


_tc_final = pl.pallas_call(
    _tc_final_body,
    out_shape=jax.ShapeDtypeStruct((1, F), jnp.float32),
)


def kernel(in_feat, edge_index, W1, b1, W2, b2):
    e = edge_index.shape[1]
    src = jnp.concatenate(
        [edge_index[0], jnp.full((E_PAD - e,), N, jnp.int32)]).reshape(NW * CH, K)
    dst = jnp.concatenate(
        [edge_index[1], jnp.full((E_PAD - e,), N, jnp.int32)]).reshape(NW * CH, K)
    x_p = jnp.pad(in_feat, ((0, NP - N), (0, 0)))

    degout_p, degin_p = _sc_degrees(src, dst)
    degout_p = degout_p.reshape(NC, NP)
    degin_p = degin_p.reshape(NC, NP)

    y, ns, nd = _tc_prep(x_p,
                         degout_p[0].reshape(NP, 1), degout_p[1].reshape(NP, 1),
                         degin_p[0].reshape(NP, 1), degin_p[1].reshape(NP, 1),
                         W1)

    agg_p, w_p = _sc_main(y, src, dst, nd.reshape(NP))
    w_p = w_p.reshape(NC, NP)

    w2p = jnp.pad(W2, ((0, 0), (0, F - C)))
    b2p = jnp.pad(b2, (0, F - C)).reshape(1, F)
    out = _tc_final(agg_p[0], agg_p[1], ns, nd,
                    w_p[0].reshape(NP, 1), w_p[1].reshape(NP, 1),
                    b1.reshape(1, F), w2p, b2p)
    return out[0, :C]


# speedup vs baseline: 7.2788x; 1.0780x over previous
import functools

import jax
import jax.numpy as jnp
from jax import lax
from jax.experimental import pallas as pl
from jax.experimental.pallas import tpu as pltpu
from jax.experimental.pallas import tpu_sc as plsc

N = 10000
F = 128
C = 40
NP = 10240
NC = 2
NS = 16
NW = NC * NS
K = 128
CH = 80
EPT = CH * K
E_PAD = NW * EPT
CH0 = 120
CH1 = 40
RPT = NP // NS


def _sc_degrees_body(src_hbm, dst_hbm, degout_hbm, degin_hbm,
                     sidx, didx, valbuf, degout_sh, degin_sh):
    cid = lax.axis_index("c")
    sid = lax.axis_index("s")
    wid = sid * NC + cid
    base = sid * RPT

    pltpu.sync_copy(src_hbm.at[pl.ds(wid * CH, CH)], sidx)
    pltpu.sync_copy(dst_hbm.at[pl.ds(wid * CH, CH)], didx)

    for i in range(K // 16):
        valbuf[pl.ds(i * 16, 16)] = jnp.zeros((16,), jnp.float32)
    for i in range(RPT // K):
        pltpu.sync_copy(valbuf, degout_sh.at[pl.ds(base + i * K, K)])
        pltpu.sync_copy(valbuf, degin_sh.at[pl.ds(base + i * K, K)])
    plsc.subcore_barrier()

    for i in range(K // 16):
        valbuf[pl.ds(i * 16, 16)] = jnp.ones((16,), jnp.float32)

    def body(j, carry):
        pltpu.sync_copy(valbuf, degout_sh.at[sidx.at[j]], add=True)
        pltpu.sync_copy(valbuf, degin_sh.at[didx.at[j]], add=True)
        return carry

    lax.fori_loop(0, CH, body, 0)
    plsc.subcore_barrier()

    pltpu.sync_copy(degout_sh.at[pl.ds(base, RPT)],
                    degout_hbm.at[pl.ds(cid * NP + base, RPT)])
    pltpu.sync_copy(degin_sh.at[pl.ds(base, RPT)],
                    degin_hbm.at[pl.ds(cid * NP + base, RPT)])


_sc_degrees = functools.partial(
    pl.kernel,
    _sc_degrees_body,
    out_type=(jax.ShapeDtypeStruct((NC * NP,), jnp.float32),
              jax.ShapeDtypeStruct((NC * NP,), jnp.float32)),
    mesh=plsc.VectorSubcoreMesh(core_axis_name="c", subcore_axis_name="s"),
    scratch_types=[
        pltpu.VMEM((CH, K), jnp.int32),
        pltpu.VMEM((CH, K), jnp.int32),
        pltpu.VMEM((K,), jnp.float32),
        pltpu.VMEM_SHARED((NP,), jnp.float32),
        pltpu.VMEM_SHARED((NP,), jnp.float32),
    ],
)()


def _sc_main_body(y_hbm, src_hbm, dst_hbm, nd_hbm, agg_hbm, w_hbm,
                  sidx, didx, rows, vals, isem0, isem1, gsem0, gsem1, vsem,
                  agg_sh, w_sh):
    cid = lax.axis_index("c")
    sid = lax.axis_index("s")
    base = sid * RPT
    isems = (isem0, isem1)
    gsems = (gsem0, gsem1)

    my_ch = jnp.where(cid == 0, CH0, CH1)
    row0 = jnp.where(cid == 0, sid * CH0, NS * CH0 + sid * CH1)

    def fire_idx(j, b, sem):
        pltpu.async_copy(src_hbm.at[row0 + j], sidx.at[b], sem)
        pltpu.async_copy(dst_hbm.at[row0 + j], didx.at[b], sem)

    def wait_idx(j, b, sem):
        pltpu.make_async_copy(src_hbm.at[row0 + j], sidx.at[b], sem).wait()
        pltpu.make_async_copy(dst_hbm.at[row0 + j], didx.at[b], sem).wait()

    def zrow(t, carry):
        rows[0, t // 8, pl.ds((t % 8) * 16, 16)] = jnp.zeros((16,), jnp.float32)
        return carry

    lax.fori_loop(0, K * 8, zrow, 0)
    for i in range(RPT // K):
        pltpu.sync_copy(rows.at[0], agg_sh.at[pl.ds(base + i * K, K)])
    for i in range(RPT // F):
        pltpu.sync_copy(rows.at[0, 0], w_sh.at[pl.ds(base + i * F, F)])
    plsc.subcore_barrier()

    fire_idx(0, 0, isem0)
    fire_idx(1, 1, isem1)
    wait_idx(0, 0, isem0)
    pltpu.async_copy(y_hbm.at[sidx.at[0]], rows.at[0], gsem0)

    def body(p, carry):
        for b in range(2):
            j = p * 2 + b
            cond_next = (j + 1 < my_ch)
            cond_pref = (j + 2 < my_ch)

            @pl.when(cond_next)
            def _():
                wait_idx(j + 1, 1 - b, isems[1 - b])
                pltpu.async_copy(y_hbm.at[sidx.at[1 - b]], rows.at[1 - b],
                                 gsems[1 - b])

            pltpu.async_copy(nd_hbm.at[didx.at[b]], vals, vsem)
            pltpu.make_async_copy(y_hbm.at[sidx.at[b]], rows.at[b],
                                  gsems[b]).wait()
            pltpu.sync_copy(rows.at[b], agg_sh.at[didx.at[b]], add=True)
            pltpu.make_async_copy(nd_hbm.at[didx.at[b]], vals, vsem).wait()
            pltpu.sync_copy(vals, w_sh.at[sidx.at[b]], add=True)

            @pl.when(cond_pref)
            def _():
                fire_idx(j + 2, b, isems[b])
        return carry

    lax.fori_loop(0, my_ch // 2, body, 0)
    plsc.subcore_barrier()

    for i in range(RPT // K):
        pltpu.sync_copy(agg_sh.at[pl.ds(base + i * K, K)],
                        agg_hbm.at[cid, pl.ds(base + i * K, K)])
    for i in range(RPT // F):
        pltpu.sync_copy(w_sh.at[pl.ds(base + i * F, F)],
                        w_hbm.at[pl.ds(cid * NP + base + i * F, F)])


_sc_main = functools.partial(
    pl.kernel,
    _sc_main_body,
    out_type=(jax.ShapeDtypeStruct((NC, NP, F), jnp.float32),
              jax.ShapeDtypeStruct((NC * NP,), jnp.float32)),
    mesh=plsc.VectorSubcoreMesh(core_axis_name="c", subcore_axis_name="s"),
    scratch_types=[
        pltpu.VMEM((2, K), jnp.int32),
        pltpu.VMEM((2, K), jnp.int32),
        pltpu.VMEM((2, K, F), jnp.float32),
        pltpu.VMEM((K,), jnp.float32),
        pltpu.SemaphoreType.DMA,
        pltpu.SemaphoreType.DMA,
        pltpu.SemaphoreType.DMA,
        pltpu.SemaphoreType.DMA,
        pltpu.SemaphoreType.DMA,
        pltpu.VMEM_SHARED((NP, F), jnp.float32),
        pltpu.VMEM_SHARED((NP,), jnp.float32),
    ],
)()


def _tc_prep_body(x_ref, do0, do1, di0, di1, w1_ref, y_ref, ns_ref, nd_ref):
    ns = lax.rsqrt(jnp.maximum(do0[...] + do1[...], 1.0))
    nd = lax.rsqrt(jnp.maximum(di0[...] + di1[...], 1.0))
    xs = x_ref[...] * ns
    y_ref[...] = jnp.dot(xs, w1_ref[...], preferred_element_type=jnp.float32)
    ns_ref[...] = ns
    nd_ref[...] = nd


_tc_prep = pl.pallas_call(
    _tc_prep_body,
    out_shape=(jax.ShapeDtypeStruct((NP, F), jnp.float32),
               jax.ShapeDtypeStruct((NP, 1), jnp.float32),
               jax.ShapeDtypeStruct((NP, 1), jnp.float32)),
)


def _tc_final_body(a0, a1, ns, nd, w0, w1, b1r, w2p, b2p, out_ref):
    agg = a0[...] + a1[...]
    h1 = jnp.maximum(nd[...] * agg + b1r[...], 0.0)
    h2s = h1 * ns[...]
    w = w0[...] + w1[...]
    rid = lax.broadcasted_iota(jnp.int32, (NP, 1), 0)
    wm = jnp.where(rid < N, w, 0.0)
    z = jnp.sum(h2s * wm, axis=0, keepdims=True)
    out_ref[...] = (jnp.dot(z * (1.0 / N), w2p[...],
                            preferred_element_type=jnp.float32) + b2p[...])


_tc_final = pl.pallas_call(
    _tc_final_body,
    out_shape=jax.ShapeDtypeStruct((1, F), jnp.float32),
)


def kernel(in_feat, edge_index, W1, b1, W2, b2):
    e = edge_index.shape[1]
    src = jnp.concatenate(
        [edge_index[0], jnp.full((E_PAD - e,), N, jnp.int32)]).reshape(NW * CH, K)
    dst = jnp.concatenate(
        [edge_index[1], jnp.full((E_PAD - e,), N, jnp.int32)]).reshape(NW * CH, K)
    x_p = jnp.pad(in_feat, ((0, NP - N), (0, 0)))

    degout_p, degin_p = _sc_degrees(src, dst)
    degout_p = degout_p.reshape(NC, NP)
    degin_p = degin_p.reshape(NC, NP)

    y, ns, nd = _tc_prep(x_p,
                         degout_p[0].reshape(NP, 1), degout_p[1].reshape(NP, 1),
                         degin_p[0].reshape(NP, 1), degin_p[1].reshape(NP, 1),
                         W1)

    agg_p, w_p = _sc_main(y, src, dst, nd.reshape(NP))
    w_p = w_p.reshape(NC, NP)

    w2p = jnp.pad(W2, ((0, 0), (0, F - C)))
    b2p = jnp.pad(b2, (0, F - C)).reshape(1, F)
    out = _tc_final(agg_p[0], agg_p[1], ns, nd,
                    w_p[0].reshape(NP, 1), w_p[1].reshape(NP, 1),
                    b1.reshape(1, F), w2p, b2p)
    return out[0, :C]
